# baseline (speedup 1.0000x reference)
import functools

import jax
import jax.numpy as jnp
from jax.experimental import pallas as pl
from jax.experimental.pallas import tpu as pltpu


def _swish(x):
    h = 0.5 * x
    return h + h * jnp.tanh(h)


def _sigmoid(x):
    return 0.5 + 0.5 * jnp.tanh(0.5 * x)


def _mbconv_body(x_ref, w_exp_ref, bn0_s_ref, bn0_b_ref, slab_ref,
                 w_sr_ref, b_sr_ref, w_se_ref, w_pr_ref, bn2_s_ref,
                 bn2_b_ref, o_ref, t0_ref, t2_ref, ea_ref, eb_ref,
                 *, K: int, H: int, W: int, residual: bool, img_per_step: int):
    HW = H * W
    Cin = w_exp_ref.shape[1]
    Cexp = slab_ref.shape[1]
    Csq = w_sr_ref.shape[1]
    K2 = K * K
    assert K == 3, "3x3 depthwise path"

    w_exp_aug = jnp.concatenate(
        [w_exp_ref[...] * bn0_s_ref[...], bn0_b_ref[...]],
        axis=1).astype(jnp.bfloat16)
    w_pr_s = (w_pr_ref[...] * bn2_s_ref[...]).astype(jnp.bfloat16)
    w_sr = w_sr_ref[...].astype(jnp.bfloat16)
    w_se = w_se_ref[...]
    b_sr = b_sr_ref[...]
    bn2_b = bn2_b_ref[...]

    slab = slab_ref[...]
    taps = [slab[i:i + 1, :].astype(jnp.bfloat16) for i in range(K2)]
    bn1_b = slab[K2:K2 + 1, :].astype(jnp.bfloat16)
    b_se = slab[K2 + 1:K2 + 2, :]

    r_idx = jax.lax.broadcasted_iota(jnp.int32, (HW, 1), 0)
    w_of_r = jax.lax.rem(r_idx, W)
    m_m1 = (w_of_r >= 1).astype(jnp.float32).astype(jnp.bfloat16)
    m_p1 = (w_of_r < W - 1).astype(jnp.float32).astype(jnp.bfloat16)
    pool_row = jnp.full((1, HW), 1.0 / HW, jnp.bfloat16)
    ones_row = jnp.ones((1, HW), jnp.bfloat16)

    half = Cexp // 2
    zpad = jnp.zeros((1, half), jnp.float32)

    def one_image(i):
        x_b = x_ref[i]
        t0 = t0_ref.at[i]
        t2 = t2_ref.at[i]
        ea = ea_ref.at[i]
        eb = eb_ref.at[i]

        x_aug = jnp.concatenate([x_b.astype(jnp.bfloat16), ones_row], axis=0)
        e_t = jax.lax.dot_general(
            x_aug, w_exp_aug,
            (((0,), (1,)), ((), ())), preferred_element_type=jnp.float32)
        es = _swish(e_t)

        ea[0:1, :] = zpad
        eb[0:1, :] = zpad
        ea[HW + 1:HW + 2, :] = zpad
        eb[HW + 1:HW + 2, :] = zpad
        ea[1:HW + 1, :] = es[:, 0:half]
        eb[1:HW + 1, :] = es[:, half:Cexp]

        sh_m1 = jnp.concatenate([ea[0:HW, :], eb[0:HW, :]], axis=1)
        sh_p1 = jnp.concatenate([ea[2:HW + 2, :], eb[2:HW + 2, :]], axis=1)
        c_m1 = sh_m1.astype(jnp.bfloat16) * m_m1
        c_p1 = sh_p1.astype(jnp.bfloat16) * m_p1
        c_0 = es.astype(jnp.bfloat16)

        def trow(dh):
            return (taps[dh * K] * c_m1 + taps[dh * K + 1] * c_0
                    + taps[dh * K + 2] * c_p1)

        t0[0:W, :] = jnp.zeros((W, Cexp), jnp.bfloat16)
        t0[W:W + HW, :] = trow(0)
        t2[W:W + HW, :] = trow(2)
        t2[W + HW:2 * W + HW, :] = jnp.zeros((W, Cexp), jnp.bfloat16)

        acc = trow(1) + t0[0:HW, :] + t2[2 * W:2 * W + HW, :]
        d = _swish(acc + bn1_b)

        pooled = jax.lax.dot_general(
            pool_row, d,
            (((1,), (0,)), ((), ())), preferred_element_type=jnp.float32)
        red = jax.lax.dot_general(
            pooled.astype(jnp.bfloat16), w_sr,
            (((1,), (0,)), ((), ())), preferred_element_type=jnp.float32)
        red = _swish(red + b_sr)
        ex = jax.lax.dot_general(
            red, w_se,
            (((1,), (1,)), ((), ())), preferred_element_type=jnp.float32)
        gate = _sigmoid(ex + b_se)

        w_pr_g = w_pr_s * gate.astype(jnp.bfloat16)
        p = jax.lax.dot_general(
            w_pr_g, d,
            (((1,), (1,)), ((), ())), preferred_element_type=jnp.float32)
        p = p + bn2_b

        if residual:
            p = p + x_b
        o_ref[i] = p

    for i in range(img_per_step):
        one_image(i)


def _mbconv_forward(x_nchw, params, *, ksize, stride):
    B, Cin, H, W = x_nchw.shape
    HW = H * W
    x = x_nchw.reshape(B, Cin, HW).astype(jnp.float32)

    K = ksize
    K2 = K * K
    Cexp = params["w_exp"].shape[0]
    Cout = params["w_pr"].shape[0]
    Csq = params["w_sr"].shape[1]
    residual = (stride == 1 and Cin == Cout)
    ips = 2 if B % 2 == 0 else 1

    taps = params["w_dw"].reshape(K2, Cexp) * params["bn1_s"].reshape(1, Cexp)
    slab = jnp.concatenate(
        [taps,
         params["bn1_b"].reshape(1, Cexp),
         params["b_se"].reshape(1, Cexp)],
        axis=0).astype(jnp.float32)
    nrows = K2 + 2

    full2 = lambda shape: pl.BlockSpec(shape, lambda b: (0, 0))
    in_specs = [
        pl.BlockSpec((ips, Cin, HW), lambda b: (b, 0, 0)),
        full2((Cexp, Cin)),
        full2((Cexp, 1)),
        full2((Cexp, 1)),
        full2((nrows, Cexp)),
        full2((Cexp, Csq)),
        full2((1, Csq)),
        full2((Cexp, Csq)),
        full2((Cout, Cexp)),
        full2((Cout, 1)),
        full2((Cout, 1)),
    ]
    out_spec = pl.BlockSpec((ips, Cout, HW), lambda b: (b, 0, 0))

    body = functools.partial(_mbconv_body, K=K, H=H, W=W, residual=residual,
                             img_per_step=ips)

    out = pl.pallas_call(
        body,
        out_shape=jax.ShapeDtypeStruct((B, Cout, HW), jnp.float32),
        grid=(B // ips,),
        in_specs=in_specs,
        out_specs=out_spec,
        scratch_shapes=[
            pltpu.VMEM((ips, HW + 2 * W, Cexp), jnp.bfloat16),
            pltpu.VMEM((ips, HW + 2 * W, Cexp), jnp.bfloat16),
            pltpu.VMEM((ips, HW + 2, Cexp // 2), jnp.float32),
            pltpu.VMEM((ips, HW + 2, Cexp // 2), jnp.float32),
        ],
        compiler_params=pltpu.CompilerParams(
            dimension_semantics=("parallel",),
            vmem_limit_bytes=60000 * 1024,
        ),
    )(x, params["w_exp"].astype(jnp.float32),
      params["bn0_s"].astype(jnp.float32),
      params["bn0_b"].astype(jnp.float32),
      slab,
      params["w_sr"].astype(jnp.float32),
      params["b_sr"].astype(jnp.float32),
      params["w_se"].astype(jnp.float32),
      params["w_pr"].astype(jnp.float32),
      params["bn2_s"].astype(jnp.float32),
      params["bn2_b"].astype(jnp.float32))

    return out.reshape(B, Cout, H, W)


def kernel(x, w_exp, bn0_s, bn0_b, w_dw, bn1_s, bn1_b, w_sr, b_sr, w_se,
           b_se, w_pr, bn2_s, bn2_b):
    params = {"w_exp": w_exp, "bn0_s": bn0_s, "bn0_b": bn0_b, "w_dw": w_dw,
              "bn1_s": bn1_s, "bn1_b": bn1_b, "w_sr": w_sr, "b_sr": b_sr,
              "w_se": w_se, "b_se": b_se, "w_pr": w_pr, "bn2_s": bn2_s,
              "bn2_b": bn2_b}
    K = int(round(w_dw.shape[0] ** 0.5))
    return _mbconv_forward(x, params, ksize=K, stride=1)
